# Initial kernel scaffold; baseline (speedup 1.0000x reference)
import jax
import jax.numpy as jnp
from jax.experimental import pallas as pl


def kernel(z, edge_index, node_positions, latent_W, latent_b, pos_W, pos_b, mlp0_W0, mlp0_b0, mlp0_W1, mlp0_b1, ln0_g, ln0_b, mlp1_W0, mlp1_b0, mlp1_W1, mlp1_b1, ln1_g, ln1_b, gnn_Wm0, gnn_bm0, gnn_Wm1, gnn_bm1, gnn_Wn0, gnn_bn0, gnn_Wn1, gnn_bn1, gnn_We0, gnn_be0, gnn_We1, gnn_be1, mlp2_W0, mlp2_b0, mlp2_W1, mlp2_b1):
    raise NotImplementedError("write your pallas kernel here")


# speedup vs baseline: 1.4442x; 1.4442x over previous
import jax
import jax.numpy as jnp
from jax import lax
from jax.experimental import pallas as pl
from jax.experimental.pallas import tpu as pltpu
from jax.experimental.pallas import tpu_sc as plsc

N = 50000
E = 800000
LD = 32
HD = 64
ED = 2
OD = 3
NL = 3

NC = 2
NS = 16
NW = NC * NS
EPAD = 802816

HALF = N // NC
ROWS_PT = 1568
HALFP = ROWS_PT * NS
TROW = HALF

_f32 = jnp.float32
_mesh = plsc.VectorSubcoreMesh(core_axis_name="c", subcore_axis_name="s")


GCH = 128
GPT = EPAD // NS


def _gather_body(P, Q, ridx, cidx, outr, outc, idx_v, rows_v, sem):
    c = lax.axis_index("c")
    s = lax.axis_index("s")
    base = s * GPT

    def run(tab, idx_hbm, out_hbm):
        def step(k, carry):
            off = pl.multiple_of(base + k * GCH, 8)
            pltpu.sync_copy(idx_hbm.at[pl.ds(off, GCH)], idx_v)
            pltpu.async_copy(tab.at[idx_v], rows_v, sem).wait()
            pltpu.sync_copy(rows_v, out_hbm.at[pl.ds(off, GCH), :])
            return carry

        lax.fori_loop(0, GPT // GCH, step, 0)

    @pl.when(c == 0)
    def _():
        run(P, ridx, outr)

    @pl.when(c == 1)
    def _():
        run(Q, cidx, outc)


_gather = pl.kernel(
    _gather_body,
    out_type=(jax.ShapeDtypeStruct((EPAD, 2 * HD), _f32),
              jax.ShapeDtypeStruct((EPAD, 2 * HD), _f32)),
    mesh=_mesh,
    scratch_types=[
        pltpu.VMEM((GCH,), jnp.int32),
        pltpu.VMEM((GCH, 2 * HD), _f32),
        pltpu.SemaphoreType.DMA,
    ],
)


PW = 128


def _gpos_body(PT, ridx, cidx, outr, outc, idx_v, rows_v, sem):
    c = lax.axis_index("c")
    s = lax.axis_index("s")
    base = s * GPT

    def run(idx_hbm, out_hbm):
        def step(k, carry):
            off = pl.multiple_of(base + k * GCH, 8)
            pltpu.sync_copy(idx_hbm.at[pl.ds(off, GCH)], idx_v)
            pltpu.async_copy(PT.at[idx_v], rows_v, sem).wait()
            pltpu.sync_copy(rows_v, out_hbm.at[pl.ds(off, GCH), :])
            return carry

        lax.fori_loop(0, GPT // GCH, step, 0)

    @pl.when(c == 0)
    def _():
        run(ridx, outr)

    @pl.when(c == 1)
    def _():
        run(cidx, outc)


_gpos = pl.kernel(
    _gpos_body,
    out_type=(jax.ShapeDtypeStruct((EPAD, PW), _f32),
              jax.ShapeDtypeStruct((EPAD, PW), _f32)),
    mesh=_mesh,
    scratch_types=[
        pltpu.VMEM((GCH,), jnp.int32),
        pltpu.VMEM((GCH, PW), _f32),
        pltpu.SemaphoreType.DMA,
    ],
)


SCH = 128
SPT = EPAD // NS

QN = N // 4
QROWS = 784
QP = QROWS * NS
QTROW = QN
MW = 2 * HD


def _scatter_body(m_hbm, col_hbm, zero_hbm, agg_hbm,
                  colbuf, idxbuf, mbuf, agg_sp):
    c = lax.axis_index("c")
    s = lax.axis_index("s")

    for q in range(2):
        qlo = c * (2 * QN) + q * QN
        obase = (c * 2 + q) * QP + s * QROWS
        pltpu.sync_copy(zero_hbm, agg_sp.at[pl.ds(s * QROWS, QROWS), :])
        plsc.subcore_barrier()

        def step(k, carry):
            off = pl.multiple_of(s * SPT + k * SCH, 8)
            pltpu.sync_copy(col_hbm.at[pl.ds(off, SCH)], colbuf)
            pltpu.sync_copy(m_hbm.at[pl.ds(off, SCH), :], mbuf)
            for v in range(SCH // 16):
                cv = colbuf[pl.ds(v * 16, 16)]
                rel = cv - qlo
                ok = (rel >= 0) & (rel < QN)
                idxbuf[0, pl.ds(v * 16, 16)] = jnp.where(ok, rel, QTROW)
            pltpu.sync_copy(mbuf, agg_sp.at[idxbuf.at[0]], add=True)
            return carry

        lax.fori_loop(0, SPT // SCH, step, 0)
        plsc.subcore_barrier()
        pltpu.sync_copy(
            agg_sp.at[pl.ds(s * QROWS, QROWS), :],
            agg_hbm.at[pl.ds(obase, QROWS), :],
        )
        plsc.subcore_barrier()


_scatter = pl.kernel(
    _scatter_body,
    out_type=jax.ShapeDtypeStruct((4 * QP, MW), _f32),
    mesh=_mesh,
    scratch_types=[
        pltpu.VMEM((SCH,), jnp.int32),
        pltpu.VMEM((1, SCH), jnp.int32),
        pltpu.VMEM((SCH, MW), _f32),
        pltpu.VMEM_SHARED((QP, MW), _f32),
    ],
)


BN = 5000
BE = 4096


def _dot(a, b):
    return jnp.dot(a, b, preferred_element_type=_f32)


def _ln_rows(x, g, b):
    m = jnp.mean(x, axis=-1, keepdims=True)
    v = jnp.mean((x - m) ** 2, axis=-1, keepdims=True)
    return (x - m) * lax.rsqrt(v + 1e-5) * g + b


def _node0_body(z, pos, latW, latb, posW, posb, W0, b0, W1, b1, g, bln, out):
    zh = _dot(z[...], latW[...]) + latb[...]
    u0 = zh + _dot(pos[...], posW[...]) + posb[...]
    h = jax.nn.relu(_dot(u0, W0[...]) + b0[...])
    u1 = _dot(h, W1[...]) + b1[...]
    out[...] = _ln_rows(u1, g[...], bln[...])


def _edge0_body(pr, pc, W0, b0, W1, b1, g, bln, out):
    e0 = pc[...][:, :ED] - pr[...][:, :ED]
    h = jax.nn.relu(_dot(e0, W0[...]) + b0[...])
    e1 = _dot(h, W1[...]) + b1[...]
    out[...] = _ln_rows(e1, g[...], bln[...])


def _pq_body(x, Wr, Wc, P_out, Q_out):
    xv = x[...]
    P_out[...] = _dot(xv, Wr[...])
    Q_out[...] = _dot(xv, Wc[...])


def _edge_body(gr, gc, e, Wc, bc, Wm1p, bm1p, We1p, be1, m_out, e_out):
    ev = e[...]
    H = jax.nn.relu(gr[...] + gc[...] + _dot(ev, Wc[...]) + bc[...])
    m_out[...] = _dot(H, Wm1p[...]) + bm1p[...]
    e_out[...] = ev + _dot(H, We1p[...]) + be1[...]


def _node_body(x, agg, Wn0a, Wn0bp, bn0, Wn1, bn1, out):
    xv = x[...]
    h = jax.nn.relu(_dot(xv, Wn0a[...]) + _dot(agg[...], Wn0bp[...])
                    + bn0[...])
    out[...] = xv + _dot(h, Wn1[...]) + bn1[...]


def _final_body(x, W0, b0, W1, b1, out):
    h = jax.nn.relu(_dot(x[...], W0[...]) + b0[...])
    out[...] = _dot(h, W1[...]) + b1[...]


def _whole(shape):
    nd = len(shape)
    return pl.BlockSpec(shape, lambda *args: (0,) * nd)


def _node0(z, pos, latW, latb, posW, posb, W0, b0, W1, b1, g, bln):
    return pl.pallas_call(
        _node0_body,
        grid=(N // BN,),
        in_specs=[
            _whole((1, LD)),
            pl.BlockSpec((BN, 2), lambda i: (i, 0)),
            _whole((LD, HD)), _whole((1, HD)),
            _whole((2, HD)), _whole((1, HD)),
            _whole((HD, HD)), _whole((1, HD)),
            _whole((HD, HD)), _whole((1, HD)),
            _whole((1, HD)), _whole((1, HD)),
        ],
        out_specs=pl.BlockSpec((BN, HD), lambda i: (i, 0)),
        out_shape=jax.ShapeDtypeStruct((N, HD), _f32),
    )(z, pos, latW, latb, posW, posb, W0, b0, W1, b1, g, bln)


def _edge0(pr, pc, W0, b0, W1, b1, g, bln):
    return pl.pallas_call(
        _edge0_body,
        grid=(EPAD // BE,),
        in_specs=[
            pl.BlockSpec((BE, PW), lambda i: (i, 0)),
            pl.BlockSpec((BE, PW), lambda i: (i, 0)),
            _whole((ED, HD)), _whole((1, HD)),
            _whole((HD, ED)), _whole((1, ED)),
            _whole((1, ED)), _whole((1, ED)),
        ],
        out_specs=pl.BlockSpec((BE, ED), lambda i: (i, 0)),
        out_shape=jax.ShapeDtypeStruct((EPAD, ED), _f32),
    )(pr, pc, W0, b0, W1, b1, g, bln)


def _pq(x, Wr, Wc):
    return pl.pallas_call(
        _pq_body,
        grid=(N // BN,),
        in_specs=[
            pl.BlockSpec((BN, HD), lambda i: (i, 0)),
            _whole((HD, 2 * HD)), _whole((HD, 2 * HD)),
        ],
        out_specs=[pl.BlockSpec((BN, 2 * HD), lambda i: (i, 0))] * 2,
        out_shape=[jax.ShapeDtypeStruct((N, 2 * HD), _f32)] * 2,
    )(x, Wr, Wc)


def _edge_layer(gr, gc, e, Wc, bc, Wm1p, bm1p, We1p, be1):
    blk = lambda w: pl.BlockSpec((BE, w), lambda i: (i, 0))
    return pl.pallas_call(
        _edge_body,
        grid=(EPAD // BE,),
        in_specs=[
            blk(2 * HD), blk(2 * HD), blk(ED),
            _whole((ED, 2 * HD)), _whole((1, 2 * HD)),
            _whole((2 * HD, MW)), _whole((1, MW)),
            _whole((2 * HD, ED)), _whole((1, ED)),
        ],
        out_specs=[blk(MW), blk(ED)],
        out_shape=[
            jax.ShapeDtypeStruct((EPAD, MW), _f32),
            jax.ShapeDtypeStruct((EPAD, ED), _f32),
        ],
    )(gr, gc, e, Wc, bc, Wm1p, bm1p, We1p, be1)


def _node_layer(x, agg, Wn0a, Wn0bp, bn0, Wn1, bn1):
    return pl.pallas_call(
        _node_body,
        grid=(N // BN,),
        in_specs=[
            pl.BlockSpec((BN, HD), lambda i: (i, 0)),
            pl.BlockSpec((BN, MW), lambda i: (i, 0)),
            _whole((HD, HD)), _whole((MW, HD)), _whole((1, HD)),
            _whole((HD, HD)), _whole((1, HD)),
        ],
        out_specs=pl.BlockSpec((BN, HD), lambda i: (i, 0)),
        out_shape=jax.ShapeDtypeStruct((N, HD), _f32),
    )(x, agg, Wn0a, Wn0bp, bn0, Wn1, bn1)


def _final(x, W0, b0, W1, b1):
    return pl.pallas_call(
        _final_body,
        grid=(N // BN,),
        in_specs=[
            pl.BlockSpec((BN, HD), lambda i: (i, 0)),
            _whole((HD, HD)), _whole((1, HD)),
            _whole((HD, OD)), _whole((1, OD)),
        ],
        out_specs=pl.BlockSpec((BN, OD), lambda i: (i, 0)),
        out_shape=jax.ShapeDtypeStruct((N, OD), _f32),
    )(x, W0, b0, W1, b1)


def kernel(z, edge_index, node_positions, latent_W, latent_b, pos_W, pos_b,
           mlp0_W0, mlp0_b0, mlp0_W1, mlp0_b1, ln0_g, ln0_b,
           mlp1_W0, mlp1_b0, mlp1_W1, mlp1_b1, ln1_g, ln1_b,
           gnn_Wm0, gnn_bm0, gnn_Wm1, gnn_bm1,
           gnn_Wn0, gnn_bn0, gnn_Wn1, gnn_bn1,
           gnn_We0, gnn_be0, gnn_We1, gnn_be1,
           mlp2_W0, mlp2_b0, mlp2_W1, mlp2_b1):
    r2 = lambda b: b.reshape(1, -1)
    pad = EPAD - E
    rowp = jnp.pad(edge_index[0], (0, pad))
    colp0 = jnp.pad(edge_index[1], (0, pad))
    colpN = jnp.pad(edge_index[1], (0, pad), constant_values=N)
    PT = jnp.pad(node_positions, ((0, 0), (0, PW - 2)))
    zrows = jnp.zeros((QROWS, MW), _f32)

    x = _node0(z, node_positions, latent_W, r2(latent_b), pos_W, r2(pos_b),
               mlp0_W0, r2(mlp0_b0), mlp0_W1, r2(mlp0_b1),
               r2(ln0_g), r2(ln0_b))

    PR, PC = _gpos(PT, rowp, colp0)
    e = _edge0(PR, PC, mlp1_W0, r2(mlp1_b0), mlp1_W1, r2(mlp1_b1),
               r2(ln1_g), r2(ln1_b))

    zpad_e = jnp.zeros((HD, ED), _f32)
    zHH = jnp.zeros((HD, HD), _f32)
    for l in range(NL):
        Wr = jnp.concatenate([gnn_Wm0[l, :HD], gnn_We0[l, :HD]], axis=1)
        Wcol = jnp.concatenate([gnn_Wm0[l, HD:2 * HD],
                                gnn_We0[l, HD:2 * HD]], axis=1)
        Wc = jnp.concatenate([gnn_Wm0[l, 2 * HD:], gnn_We0[l, 2 * HD:]],
                             axis=1)
        bc = jnp.concatenate([gnn_bm0[l], gnn_be0[l]]).reshape(1, -1)
        Wm1p = jnp.concatenate(
            [jnp.concatenate([gnn_Wm1[l], zHH], axis=1),
             jnp.zeros((HD, MW), _f32)], axis=0)
        bm1p = jnp.concatenate(
            [gnn_bm1[l], jnp.zeros((HD,), _f32)]).reshape(1, -1)
        We1p = jnp.concatenate([zpad_e, gnn_We1[l]], axis=0)
        Wn0bp = jnp.concatenate([gnn_Wn0[l, HD:], zHH], axis=0)

        P, Q = _pq(x, Wr, Wcol)
        Gr, Gc = _gather(P, Q, rowp, colp0)
        m, e = _edge_layer(Gr, Gc, e, Wc, bc, Wm1p, bm1p,
                           We1p, r2(gnn_be1[l]))
        aggr = _scatter(m, colpN, zrows)
        agg = jnp.concatenate(
            [lax.slice_in_dim(aggr, b * QP, b * QP + QN) for b in range(4)],
            axis=0)
        x = _node_layer(x, agg, gnn_Wn0[l, :HD], Wn0bp,
                        r2(gnn_bn0[l]), gnn_Wn1[l], r2(gnn_bn1[l]))

    return _final(x, mlp2_W0, r2(mlp2_b0), mlp2_W1, r2(mlp2_b1))
